# baseline (speedup 1.0000x reference)
import functools

import jax
import jax.numpy as jnp
from jax import lax
from jax.experimental import pallas as pl
from jax.experimental.pallas import tpu as pltpu
from jax.experimental.pallas import tpu_sc as plsc

USER_DIM = 32
MOVIE_DIM = 32
CAT_DIM = USER_DIM + MOVIE_DIM
OUT_W = 128
WAVE = 32
LANES = 16


def _issue_wave(table, g_v, grp, jbase, sem):
    def issue(j, carry):
        g = g_v[pl.ds(jbase + j, LANES)][0]
        pltpu.async_copy(table.at[pl.ds(g * 8, 8)], grp.at[j], sem)
        return carry

    lax.fori_loop(0, WAVE, issue, 0)


def _drain_wave(table, grp, sem):
    def drain(j, carry):
        pltpu.make_async_copy(table.at[pl.ds(0, 8)], grp.at[j], sem).wait()
        return carry

    lax.fori_loop(0, WAVE, drain, 0)


def _extract_wave(grp, sub_v, buf, jbase, col0):
    iota = lax.iota(jnp.int32, LANES)

    def one(jl, carry):
        j = jbase + jl
        jv = jnp.full((LANES,), j, jnp.int32)
        subj = plsc.load_gather(sub_v, [jv])
        jlv = jnp.full((LANES,), jl, jnp.int32)
        lo = plsc.load_gather(grp, [jlv, subj, iota])
        hi = plsc.load_gather(grp, [jlv, subj, iota + LANES])
        buf[pl.ds(jl * OUT_W + col0, LANES)] = lo
        buf[pl.ds(jl * OUT_W + col0 + LANES, LANES)] = hi
        return carry

    lax.fori_loop(0, WAVE, one, 0)


def _gather_body(b_per_w, nc, x1_hbm, x2_hbm, ue_hbm, me_hbm, out_hbm,
                 idx1_v, idx2_v, sub1_v, sub2_v,
                 grp_u, grp_m, buf, sem):
    wid = lax.axis_index("s") * nc + lax.axis_index("c")
    base = wid * b_per_w
    pltpu.sync_copy(x1_hbm.at[pl.ds(base, b_per_w)],
                    idx1_v.at[pl.ds(0, b_per_w)])
    pltpu.sync_copy(x2_hbm.at[pl.ds(base, b_per_w)],
                    idx2_v.at[pl.ds(0, b_per_w)])

    def split(k, carry):
        s = k * LANES
        i1 = idx1_v[pl.ds(s, LANES)]
        i2 = idx2_v[pl.ds(s, LANES)]
        sub1_v[pl.ds(s, LANES)] = lax.bitwise_and(i1, 7)
        sub2_v[pl.ds(s, LANES)] = lax.bitwise_and(i2, 7)
        idx1_v[pl.ds(s, LANES)] = lax.shift_right_logical(i1, 3)
        idx2_v[pl.ds(s, LANES)] = lax.shift_right_logical(i2, 3)
        return carry

    lax.fori_loop(0, b_per_w // LANES, split, 0)

    for w in range(b_per_w // WAVE):
        jbase = w * WAVE
        _issue_wave(ue_hbm, idx1_v, grp_u, jbase, sem)
        _issue_wave(me_hbm, idx2_v, grp_m, jbase, sem)
        _drain_wave(ue_hbm, grp_u, sem)
        _extract_wave(grp_u, sub1_v, buf, jbase, 0)
        _drain_wave(me_hbm, grp_m, sem)
        _extract_wave(grp_m, sub2_v, buf, jbase, USER_DIM)
        pltpu.sync_copy(
            buf, out_hbm.at[pl.ds((base + jbase) * OUT_W, WAVE * OUT_W)])


def _mlp_body(x_ref, w1_ref, b1_ref, w2_ref, b2_ref, o_ref):
    x = x_ref[...][:, :CAT_DIM]
    h = jnp.dot(x, w1_ref[...],
                preferred_element_type=jnp.float32) + b1_ref[...]
    h = jnp.maximum(h, 0.0)
    o = jnp.dot(h, w2_ref[...],
                preferred_element_type=jnp.float32) + b2_ref[...]
    o_ref[...] = jax.nn.sigmoid(o)


def kernel(x1, x2, user_embed, movie_embed, fc1_w, fc1_b, fc2_w, fc2_b):
    B = x1.shape[0]
    info = plsc.get_sparse_core_info()
    nc, ns = info.num_cores, info.num_subcores
    nw = nc * ns
    b_per_w = B // nw

    x1i = x1.astype(jnp.int32)
    x2i = x2.astype(jnp.int32)

    gather = pl.kernel(
        functools.partial(_gather_body, b_per_w, nc),
        out_type=jax.ShapeDtypeStruct((B * OUT_W,), jnp.float32),
        mesh=plsc.VectorSubcoreMesh(core_axis_name="c", subcore_axis_name="s"),
        scratch_types=[
            pltpu.VMEM((b_per_w + LANES,), jnp.int32),
            pltpu.VMEM((b_per_w + LANES,), jnp.int32),
            pltpu.VMEM((b_per_w,), jnp.int32),
            pltpu.VMEM((b_per_w,), jnp.int32),
            pltpu.VMEM((WAVE, 8, USER_DIM), jnp.float32),
            pltpu.VMEM((WAVE, 8, MOVIE_DIM), jnp.float32),
            pltpu.VMEM((WAVE * OUT_W,), jnp.float32),
            pltpu.SemaphoreType.DMA,
        ],
        compiler_params=pltpu.CompilerParams(needs_layout_passes=False,
                                             use_tc_tiling_on_sc=True),
    )
    xflat = gather(x1i, x2i, user_embed, movie_embed)
    x = xflat.reshape(B, OUT_W)

    hidden = fc1_w.shape[0]
    hp = 128
    w1t = jnp.zeros((CAT_DIM, hp), jnp.float32).at[:, :hidden].set(fc1_w.T)
    b1 = jnp.zeros((1, hp), jnp.float32).at[:, :hidden].set(fc1_b[None, :])
    w2t = jnp.zeros((hp, 1), jnp.float32).at[:hidden, :].set(fc2_w.T)
    b2 = fc2_b.reshape(1, 1)

    blk = 2048
    grid = (B // blk,)
    out = pl.pallas_call(
        _mlp_body,
        grid=grid,
        in_specs=[
            pl.BlockSpec((blk, OUT_W), lambda i: (i, 0)),
            pl.BlockSpec((CAT_DIM, hp), lambda i: (0, 0)),
            pl.BlockSpec((1, hp), lambda i: (0, 0)),
            pl.BlockSpec((hp, 1), lambda i: (0, 0)),
            pl.BlockSpec((1, 1), lambda i: (0, 0)),
        ],
        out_specs=pl.BlockSpec((blk, 1), lambda i: (i, 0)),
        out_shape=jax.ShapeDtypeStruct((B, 1), jnp.float32),
        compiler_params=pltpu.CompilerParams(
            dimension_semantics=("arbitrary",)),
    )(x, w1t, b1, w2t, b2)
    return out


# speedup vs baseline: 1.5794x; 1.5794x over previous
import functools

import jax
import jax.numpy as jnp
from jax import lax
from jax.experimental import pallas as pl
from jax.experimental.pallas import tpu as pltpu
from jax.experimental.pallas import tpu_sc as plsc

USER_DIM = 32
MOVIE_DIM = 32
CAT_DIM = USER_DIM + MOVIE_DIM
OUT_W = 128
WAVE = 16
LANES = 16


def _gather_body(b_per_w, nc, x1_hbm, x2_hbm, ue_hbm, me_hbm, out_hbm,
                 idx1_v, idx2_v, sub1_v, sub2_v,
                 grp_u, grp_m, buf, sem_u, sem_m, sem_w):
    wid = lax.axis_index("s") * nc + lax.axis_index("c")
    base = wid * b_per_w
    pltpu.sync_copy(x1_hbm.at[pl.ds(base, b_per_w)], idx1_v)
    pltpu.sync_copy(x2_hbm.at[pl.ds(base, b_per_w)], idx2_v)

    def split(k, carry):
        s = k * LANES
        i1 = idx1_v[pl.ds(s, LANES)]
        i2 = idx2_v[pl.ds(s, LANES)]
        sub1_v[pl.ds(s, LANES)] = lax.bitwise_and(i1, 7)
        sub2_v[pl.ds(s, LANES)] = lax.bitwise_and(i2, 7)
        idx1_v[pl.ds(s, LANES)] = lax.shift_right_logical(i1, 3)
        idx2_v[pl.ds(s, LANES)] = lax.shift_right_logical(i2, 3)
        return carry

    lax.fori_loop(0, b_per_w // LANES, split, 0)

    iota = lax.iota(jnp.int32, LANES)

    def issue(w, slot):
        jbase = w * WAVE
        i1 = idx1_v[pl.ds(jbase, LANES)]
        i2 = idx2_v[pl.ds(jbase, LANES)]
        for t in range(WAVE):
            pltpu.async_copy(ue_hbm.at[i1[t]], grp_u.at[slot, t],
                             sem_u.at[slot])
        for t in range(WAVE):
            pltpu.async_copy(me_hbm.at[i2[t]], grp_m.at[slot, t],
                             sem_m.at[slot])

    def extract(grp, sub_v, jbase, slot, col0):
        def one(jl, carry):
            jv = jnp.full((LANES,), jbase + jl, jnp.int32)
            subj = plsc.load_gather(sub_v, [jv])
            jlv = jnp.full((LANES,), jl, jnp.int32)
            lo = plsc.load_gather(grp, [jlv, subj, iota])
            hi = plsc.load_gather(grp, [jlv, subj, iota + LANES])
            buf[slot, pl.ds(jl * OUT_W + col0, LANES)] = lo
            buf[slot, pl.ds(jl * OUT_W + col0 + LANES, LANES)] = hi
            return carry

        lax.fori_loop(0, WAVE, one, 0)

    def process(w, slot):
        @pl.when(w >= 2)
        def _():
            pltpu.make_async_copy(out_hbm.at[pl.ds(0, WAVE * OUT_W)],
                                  buf.at[slot], sem_w.at[slot]).wait()

        jbase = w * WAVE
        pltpu.make_async_copy(ue_hbm.at[pl.ds(0, WAVE)], grp_u.at[slot],
                              sem_u.at[slot]).wait()
        extract(grp_u.at[slot], sub1_v, jbase, slot, 0)
        pltpu.make_async_copy(me_hbm.at[pl.ds(0, WAVE)], grp_m.at[slot],
                              sem_m.at[slot]).wait()
        extract(grp_m.at[slot], sub2_v, jbase, slot, USER_DIM)
        pltpu.async_copy(
            buf.at[slot],
            out_hbm.at[pl.ds((base + jbase) * OUT_W, WAVE * OUT_W)],
            sem_w.at[slot])

    n_waves = b_per_w // WAVE
    issue(0, 0)

    def wave(w, carry):
        slot = lax.rem(w, 2)
        nslot = lax.rem(w + 1, 2)
        issue(w + 1, nslot)
        process(w, slot)
        return carry

    lax.fori_loop(0, n_waves - 1, wave, 0)
    process(n_waves - 1, lax.rem(n_waves - 1, 2))

    pltpu.make_async_copy(out_hbm.at[pl.ds(0, WAVE * OUT_W)], buf.at[0],
                          sem_w.at[0]).wait()
    pltpu.make_async_copy(out_hbm.at[pl.ds(0, WAVE * OUT_W)], buf.at[1],
                          sem_w.at[1]).wait()


def _mlp_body(x_ref, w1_ref, b1_ref, w2_ref, b2_ref, o_ref):
    x = x_ref[...][:, :CAT_DIM]
    h = jnp.dot(x, w1_ref[...],
                preferred_element_type=jnp.float32) + b1_ref[...]
    h = jnp.maximum(h, 0.0)
    o = jnp.dot(h, w2_ref[...],
                preferred_element_type=jnp.float32) + b2_ref[...]
    o_ref[...] = jax.nn.sigmoid(o)


def kernel(x1, x2, user_embed, movie_embed, fc1_w, fc1_b, fc2_w, fc2_b):
    B = x1.shape[0]
    info = plsc.get_sparse_core_info()
    nc, ns = info.num_cores, info.num_subcores
    nw = nc * ns
    b_per_w = B // nw

    x1i = x1.astype(jnp.int32)
    x2i = x2.astype(jnp.int32)
    nu, nm = user_embed.shape[0], movie_embed.shape[0]
    ue3 = user_embed.reshape(nu // 8, 8, USER_DIM)
    me3 = movie_embed.reshape(nm // 8, 8, MOVIE_DIM)

    gather = pl.kernel(
        functools.partial(_gather_body, b_per_w, nc),
        out_type=jax.ShapeDtypeStruct((B * OUT_W,), jnp.float32),
        mesh=plsc.VectorSubcoreMesh(core_axis_name="c", subcore_axis_name="s"),
        scratch_types=[
            pltpu.VMEM((b_per_w,), jnp.int32),
            pltpu.VMEM((b_per_w,), jnp.int32),
            pltpu.VMEM((b_per_w,), jnp.int32),
            pltpu.VMEM((b_per_w,), jnp.int32),
            pltpu.VMEM((2, WAVE, 8, USER_DIM), jnp.float32),
            pltpu.VMEM((2, WAVE, 8, MOVIE_DIM), jnp.float32),
            pltpu.VMEM((2, WAVE * OUT_W), jnp.float32),
            pltpu.SemaphoreType.DMA((2,)),
            pltpu.SemaphoreType.DMA((2,)),
            pltpu.SemaphoreType.DMA((2,)),
        ],
        compiler_params=pltpu.CompilerParams(needs_layout_passes=False),
    )
    xflat = gather(x1i, x2i, ue3, me3)
    x = xflat.reshape(B, OUT_W)

    hidden = fc1_w.shape[0]
    hp = 128
    w1t = jnp.zeros((CAT_DIM, hp), jnp.float32).at[:, :hidden].set(fc1_w.T)
    b1 = jnp.zeros((1, hp), jnp.float32).at[:, :hidden].set(fc1_b[None, :])
    w2t = jnp.zeros((hp, 1), jnp.float32).at[:hidden, :].set(fc2_w.T)
    b2 = fc2_b.reshape(1, 1)

    blk = 2048
    grid = (B // blk,)
    out = pl.pallas_call(
        _mlp_body,
        grid=grid,
        in_specs=[
            pl.BlockSpec((blk, OUT_W), lambda i: (i, 0)),
            pl.BlockSpec((CAT_DIM, hp), lambda i: (0, 0)),
            pl.BlockSpec((1, hp), lambda i: (0, 0)),
            pl.BlockSpec((hp, 1), lambda i: (0, 0)),
            pl.BlockSpec((1, 1), lambda i: (0, 0)),
        ],
        out_specs=pl.BlockSpec((blk, 1), lambda i: (i, 0)),
        out_shape=jax.ShapeDtypeStruct((B, 1), jnp.float32),
        compiler_params=pltpu.CompilerParams(
            dimension_semantics=("arbitrary",)),
    )(x, w1t, b1, w2t, b2)
    return out
